# baseline (speedup 1.0000x reference)
import jax
import jax.numpy as jnp
from jax.experimental import pallas as pl
from jax.experimental.pallas import tpu as pltpu

_HIDDEN = 4096
_EXPERTS = 64
_K = 8
_BT = 1024
_BK = 1024
_NK = _HIDDEN // _BK


def _gate_kernel(x_ref, wt_ref, w_out_ref, i_out_ref, acc_ref):
    k = pl.program_id(1)
    part = jnp.dot(x_ref[...], wt_ref[...], preferred_element_type=jnp.float32)

    @pl.when(k == 0)
    def _init():
        acc_ref[...] = part

    @pl.when(k > 0)
    def _accum():
        acc_ref[...] += part

    @pl.when(k == _NK - 1)
    def _finish():
        logits = acc_ref[...]
        m = jnp.max(logits, axis=-1, keepdims=True)
        e = jnp.exp(logits - m)
        z = jnp.sum(e, axis=-1, keepdims=True)
        scores = e / z

        lane = jax.lax.broadcasted_iota(jnp.int32, scores.shape, 1)
        s = scores
        vals = []
        idxs = []
        for _ in range(_K):
            v = jnp.max(s, axis=-1)
            i = jnp.argmax(s, axis=-1).astype(jnp.int32)
            vals.append(v)
            idxs.append(i)
            s = jnp.where(lane == i[:, None], -1.0, s)

        w = jnp.stack(vals, axis=1)
        ii = jnp.stack(idxs, axis=1)
        denom = jnp.sum(w, axis=-1, keepdims=True) + 1e-20
        w_out_ref[...] = w / denom
        i_out_ref[...] = ii


def kernel(hidden_states, weight):
    b, s, h = hidden_states.shape
    x = hidden_states.reshape(-1, h).astype(jnp.float32)
    n = x.shape[0]
    wt = weight.astype(jnp.float32).T

    grid = (n // _BT, _NK)
    topk_w, topk_i = pl.pallas_call(
        _gate_kernel,
        grid=grid,
        in_specs=[
            pl.BlockSpec((_BT, _BK), lambda i, k: (i, k)),
            pl.BlockSpec((_BK, _EXPERTS), lambda i, k: (k, 0)),
        ],
        out_specs=[
            pl.BlockSpec((_BT, _K), lambda i, k: (i, 0)),
            pl.BlockSpec((_BT, _K), lambda i, k: (i, 0)),
        ],
        out_shape=[
            jax.ShapeDtypeStruct((n, _K), jnp.float32),
            jax.ShapeDtypeStruct((n, _K), jnp.int32),
        ],
        scratch_shapes=[pltpu.VMEM((_BT, _EXPERTS), jnp.float32)],
        compiler_params=pltpu.CompilerParams(
            dimension_semantics=("arbitrary", "arbitrary"),
            vmem_limit_bytes=100 * 1024 * 1024,
        ),
    )(x, wt)
    return topk_w, topk_i


# speedup vs baseline: 1.4369x; 1.4369x over previous
import jax
import jax.numpy as jnp
from jax.experimental import pallas as pl
from jax.experimental.pallas import tpu as pltpu

_HIDDEN = 4096
_EXPERTS = 64
_K = 8
_BT = 1024


def _gate_kernel(x_ref, wt_ref, w_out_ref, i_out_ref):
    x = x_ref[...]
    wt = wt_ref[...]
    logits = jnp.dot(x, wt, preferred_element_type=jnp.float32)
    m = jnp.max(logits, axis=-1, keepdims=True)
    e = jnp.exp(logits - m)
    z = jnp.sum(e, axis=-1, keepdims=True)
    scores = e / z

    lane = jax.lax.broadcasted_iota(jnp.int32, scores.shape, 1)
    s = scores
    vals = []
    idxs = []
    for _ in range(_K):
        v = jnp.max(s, axis=-1)
        i = jnp.argmax(s, axis=-1).astype(jnp.int32)
        vals.append(v)
        idxs.append(i)
        s = jnp.where(lane == i[:, None], -1.0, s)

    w = jnp.stack(vals, axis=1)
    ii = jnp.stack(idxs, axis=1)
    denom = jnp.sum(w, axis=-1, keepdims=True) + 1e-20
    w_out_ref[...] = w / denom
    i_out_ref[...] = ii


def kernel(hidden_states, weight):
    b, s, h = hidden_states.shape
    x = hidden_states.reshape(-1, h).astype(jnp.float32)
    n = x.shape[0]
    wt = weight.astype(jnp.float32).T

    grid = (n // _BT,)
    topk_w, topk_i = pl.pallas_call(
        _gate_kernel,
        grid=grid,
        in_specs=[
            pl.BlockSpec((_BT, h), lambda i: (i, 0)),
            pl.BlockSpec((h, _EXPERTS), lambda i: (0, 0)),
        ],
        out_specs=[
            pl.BlockSpec((_BT, _K), lambda i: (i, 0)),
            pl.BlockSpec((_BT, _K), lambda i: (i, 0)),
        ],
        out_shape=[
            jax.ShapeDtypeStruct((n, _K), jnp.float32),
            jax.ShapeDtypeStruct((n, _K), jnp.int32),
        ],
        compiler_params=pltpu.CompilerParams(
            dimension_semantics=("parallel",),
            vmem_limit_bytes=100 * 1024 * 1024,
        ),
    )(x, wt)
    return topk_w, topk_i
